# baseline (speedup 1.0000x reference)
import functools

import jax
import jax.numpy as jnp
import numpy as np
from jax import lax
from jax.experimental import pallas as pl
from jax.experimental.pallas import tpu as pltpu
from jax.experimental.pallas import tpu_sc as plsc

B, N = 128, 106
OUT_W = 512
UPSCALE = 4
STRIDE = 4
SIGMA = 2.0
RADIUS = int(SIGMA * 3)
KSIZE = 2 * RADIUS + 1

ROWS = B * N
WORKERS = 32
ROWS_PER_W = 2 * ROWS // WORKERS
GROUPS = ROWS_PER_W // 16
GW = 16 * OUT_W

_GVALS = np.exp(-((np.arange(KSIZE) - RADIUS) ** 2.0) / (2.0 * SIGMA ** 2)).astype(np.float32)


def _sc_scatter(pos, oth):
    mesh = plsc.VectorSubcoreMesh(core_axis_name="c", subcore_axis_name="s")

    @functools.partial(
        pl.kernel,
        out_type=[
            jax.ShapeDtypeStruct((ROWS * OUT_W,), jnp.float32),
            jax.ShapeDtypeStruct((ROWS * OUT_W,), jnp.float32),
        ],
        mesh=mesh,
        scratch_types=[
            pltpu.VMEM((2 * GW,), jnp.float32),
            pltpu.VMEM((ROWS_PER_W,), jnp.int32),
            pltpu.VMEM((ROWS_PER_W,), jnp.int32),
            pltpu.SemaphoreType.DMA,
            pltpu.SemaphoreType.DMA,
        ],
        compiler_params=pltpu.CompilerParams(needs_layout_passes=False),
    )
    def k(pos_hbm, oth_hbm, out_x, out_y, buf, posv, othv, sem0, sem1):
        wid = lax.axis_index("s") * 2 + lax.axis_index("c")
        base_row = wid * ROWS_PER_W
        is_x = base_row < ROWS
        local_word = jnp.where(is_x, base_row, base_row - ROWS) * OUT_W

        pltpu.sync_copy(pos_hbm.at[pl.ds(base_row, ROWS_PER_W)], posv)
        pltpu.sync_copy(oth_hbm.at[pl.ds(base_row, ROWS_PER_W)], othv)

        zeros16 = jnp.zeros((16,), jnp.float32)

        def zero_body(i, c):
            buf[pl.ds(i * 16, 16)] = zeros16
            return c

        lax.fori_loop(0, 2 * GW // 16, zero_body, 0)

        lanes = lax.iota(jnp.int32, 16)
        gvecs = [jnp.full((16,), float(v), jnp.float32) for v in _GVALS]
        zvecs = [zeros16] * KSIZE

        def scatter_group(g, s16, vals):
            p = posv[pl.ds(g * 16, 16)]
            o = othv[pl.ds(g * 16, 16)]
            ul = p - RADIUS
            br = p + RADIUS + 1
            ulo = o - RADIUS
            bro = o + RADIUS + 1
            in_ul = (ul >= 0) & (ul <= OUT_W) & (ulo >= 0) & (ulo <= OUT_W)
            in_br = (br >= 0) & (br <= OUT_W) & (bro >= 0) & (bro <= OUT_W)
            valid = in_ul | in_br
            lane_word = (lanes + s16) * OUT_W
            for j in range(KSIZE):
                col = ul + j
                m = valid & (col >= 0) & (col < OUT_W)
                plsc.store_scatter(buf, [lane_word + col], vals[j], mask=m)

        def body(g, c):
            odd = lax.rem(g, 2)
            even = odd == 0
            s16 = odd * 16

            @pl.when(g >= 2)
            def _restore():
                @pl.when(even)
                def _w0():
                    pltpu.make_async_copy(
                        buf.at[pl.ds(0, GW)], out_x.at[pl.ds(0, GW)], sem0).wait()

                @pl.when(jnp.logical_not(even))
                def _w1():
                    pltpu.make_async_copy(
                        buf.at[pl.ds(GW, GW)], out_x.at[pl.ds(0, GW)], sem1).wait()

                scatter_group(g - 2, s16, zvecs)

            scatter_group(g, s16, gvecs)

            dst = local_word + g * GW

            @pl.when(is_x & even)
            def _dx0():
                pltpu.async_copy(buf.at[pl.ds(0, GW)], out_x.at[pl.ds(dst, GW)], sem0)

            @pl.when(is_x & jnp.logical_not(even))
            def _dx1():
                pltpu.async_copy(buf.at[pl.ds(GW, GW)], out_x.at[pl.ds(dst, GW)], sem1)

            @pl.when(jnp.logical_not(is_x) & even)
            def _dy0():
                pltpu.async_copy(buf.at[pl.ds(0, GW)], out_y.at[pl.ds(dst, GW)], sem0)

            @pl.when(jnp.logical_not(is_x) & jnp.logical_not(even))
            def _dy1():
                pltpu.async_copy(buf.at[pl.ds(GW, GW)], out_y.at[pl.ds(dst, GW)], sem1)

            return c

        lax.fori_loop(0, GROUPS, body, 0)

        pltpu.make_async_copy(buf.at[pl.ds(GW, GW)], out_x.at[pl.ds(0, GW)], sem1).wait()
        pltpu.make_async_copy(buf.at[pl.ds(0, GW)], out_x.at[pl.ds(0, GW)], sem0).wait()

    return k(pos, oth)


def kernel(lmks):
    li = (lmks * UPSCALE / STRIDE).astype(jnp.int32)
    x = li[..., 0].reshape(-1)
    y = li[..., 1].reshape(-1)
    pos = jnp.concatenate([x, y])
    oth = jnp.concatenate([y, x])
    ox, oy = _sc_scatter(pos, oth)
    return ox.reshape(B, N, OUT_W), oy.reshape(B, N, OUT_W)


# speedup vs baseline: 1.2618x; 1.2618x over previous
import functools

import jax
import jax.numpy as jnp
import numpy as np
from jax import lax
from jax.experimental import pallas as pl
from jax.experimental.pallas import tpu as pltpu
from jax.experimental.pallas import tpu_sc as plsc

B, N = 128, 106
OUT_W = 512
UPSCALE = 4
STRIDE = 4
SIGMA = 2.0
RADIUS = int(SIGMA * 3)
KSIZE = 2 * RADIUS + 1

ROWS = B * N
WORKERS = 32
ROWS_PER_W = 2 * ROWS // WORKERS
SLABS_PER_W = ROWS_PER_W // N
GPS = (N + 15) // 16

_GVALS = np.exp(-((np.arange(KSIZE) - RADIUS) ** 2.0) / (2.0 * SIGMA ** 2)).astype(np.float32)


def _sc_scatter(pos, oth):
    mesh = plsc.VectorSubcoreMesh(core_axis_name="c", subcore_axis_name="s")

    @functools.partial(
        pl.kernel,
        out_type=[
            jax.ShapeDtypeStruct((B, N, OUT_W), jnp.float32),
            jax.ShapeDtypeStruct((B, N, OUT_W), jnp.float32),
        ],
        mesh=mesh,
        scratch_types=[
            pltpu.VMEM((2, N, OUT_W), jnp.float32),
            pltpu.VMEM((ROWS_PER_W + 16,), jnp.int32),
            pltpu.VMEM((ROWS_PER_W + 16,), jnp.int32),
            pltpu.SemaphoreType.DMA,
            pltpu.SemaphoreType.DMA,
        ],
        compiler_params=pltpu.CompilerParams(needs_layout_passes=False),
    )
    def k(pos_hbm, oth_hbm, out_x, out_y, buf, posv, othv, sem0, sem1):
        wid = lax.axis_index("s") * 2 + lax.axis_index("c")
        base_row = wid * ROWS_PER_W
        is_x = base_row < ROWS
        b_base = jnp.where(is_x, wid, wid - 16) * SLABS_PER_W
        sems = (sem0, sem1)
        outs = (out_x, out_y)

        pltpu.sync_copy(pos_hbm.at[pl.ds(base_row, ROWS_PER_W)],
                        posv.at[pl.ds(0, ROWS_PER_W)])
        pltpu.sync_copy(oth_hbm.at[pl.ds(base_row, ROWS_PER_W)],
                        othv.at[pl.ds(0, ROWS_PER_W)])

        zeros16 = jnp.zeros((16,), jnp.float32)

        def zero_body(i, c):
            sl = i // N
            r = i - sl * N
            for cc in range(OUT_W // 16):
                buf[sl, r, pl.ds(cc * 16, 16)] = zeros16
            return c

        lax.fori_loop(0, 2 * N, zero_body, 0)

        lanes = lax.iota(jnp.int32, 16)
        gvecs = [jnp.full((16,), float(v), jnp.float32) for v in _GVALS]
        zvecs = [zeros16] * KSIZE

        def scatter_group(si, g, slot, vals):
            n0 = g * 16
            woff = si * N + n0
            p = posv[pl.ds(woff, 16)]
            o = othv[pl.ds(woff, 16)]
            ul = p - RADIUS
            br = p + RADIUS + 1
            ulo = o - RADIUS
            bro = o + RADIUS + 1
            in_ul = (ul >= 0) & (ul <= OUT_W) & (ulo >= 0) & (ulo <= OUT_W)
            in_br = (br >= 0) & (br <= OUT_W) & (bro >= 0) & (bro <= OUT_W)
            row = n0 + lanes
            valid = (in_ul | in_br) & (row < N)
            slotv = jnp.full((16,), slot, jnp.int32)
            for j in range(KSIZE):
                col = ul + j
                m = valid & (col >= 0) & (col < OUT_W)
                plsc.store_scatter(buf, [slotv, row, col], vals[j], mask=m)

        def body(si, c):
            slot = lax.rem(si, 2)

            @pl.when(si >= 2)
            def _drain_and_restore():
                for sl in range(2):
                    @pl.when(slot == sl)
                    def _w(sl=sl):
                        pltpu.make_async_copy(
                            buf.at[sl], out_x.at[0], sems[sl]).wait()

                for g in range(GPS):
                    scatter_group(si - 2, g, slot, zvecs)

            for g in range(GPS):
                scatter_group(si, g, slot, gvecs)

            bcur = b_base + si
            for xv in range(2):
                for sl in range(2):
                    cond = ((is_x if xv == 0 else jnp.logical_not(is_x))
                            & (slot == sl))

                    @pl.when(cond)
                    def _d(xv=xv, sl=sl):
                        pltpu.async_copy(
                            buf.at[sl], outs[xv].at[bcur], sems[sl])

            return c

        lax.fori_loop(0, SLABS_PER_W, body, 0)

        pltpu.make_async_copy(buf.at[0], out_x.at[0], sem0).wait()
        pltpu.make_async_copy(buf.at[1], out_x.at[0], sem1).wait()

    return k(pos, oth)


def kernel(lmks):
    li = (lmks * UPSCALE / STRIDE).astype(jnp.int32)
    x = li[..., 0].reshape(-1)
    y = li[..., 1].reshape(-1)
    pos = jnp.concatenate([x, y])
    oth = jnp.concatenate([y, x])
    return tuple(_sc_scatter(pos, oth))


# speedup vs baseline: 3.2210x; 2.5527x over previous
import functools

import jax
import jax.numpy as jnp
import numpy as np
from jax import lax
from jax.experimental import pallas as pl
from jax.experimental.pallas import tpu as pltpu
from jax.experimental.pallas import tpu_sc as plsc

B, N = 128, 106
OUT_W = 512
UPSCALE = 4
STRIDE = 4
SIGMA = 2.0
RADIUS = int(SIGMA * 3)
KSIZE = 2 * RADIUS + 1

BL = 8
NL = 14
CHUNKS = (N + NL - 1) // NL
GPC = NL * BL // 16
NLAST = N - NL * (CHUNKS - 1)

_GVALS = np.exp(-((np.arange(KSIZE) - RADIUS) ** 2.0) / (2.0 * SIGMA ** 2)).astype(np.float32)


def _sc_scatter(xpl, ypl):
    mesh = plsc.VectorSubcoreMesh(core_axis_name="c", subcore_axis_name="s")

    @functools.partial(
        pl.kernel,
        out_type=[
            jax.ShapeDtypeStruct((N, B, OUT_W), jnp.float32),
            jax.ShapeDtypeStruct((N, B, OUT_W), jnp.float32),
        ],
        mesh=mesh,
        scratch_types=[
            pltpu.VMEM((2, NL, BL, OUT_W), jnp.float32),
            pltpu.VMEM((BL, N), jnp.float32),
            pltpu.VMEM((BL, N), jnp.float32),
            pltpu.SemaphoreType.DMA,
            pltpu.SemaphoreType.DMA,
        ],
        compiler_params=pltpu.CompilerParams(needs_layout_passes=False),
    )
    def k(x_hbm, y_hbm, out_x, out_y, buf, posv, othv, sem0, sem1):
        wid = lax.axis_index("s") * 2 + lax.axis_index("c")
        is_x = wid < 16
        b0 = jnp.where(is_x, wid, wid - 16) * BL
        sems = (sem0, sem1)
        outs = (out_x, out_y)

        @pl.when(is_x)
        def _sx():
            pltpu.sync_copy(x_hbm.at[pl.ds(b0, BL)], posv)
            pltpu.sync_copy(y_hbm.at[pl.ds(b0, BL)], othv)

        @pl.when(jnp.logical_not(is_x))
        def _sy():
            pltpu.sync_copy(y_hbm.at[pl.ds(b0, BL)], posv)
            pltpu.sync_copy(x_hbm.at[pl.ds(b0, BL)], othv)

        zeros16 = jnp.zeros((16,), jnp.float32)

        def zero_half(sl):
            def zbody(i, c):
                nl = i // BL
                bl = i - nl * BL
                for cc in range(OUT_W // 16):
                    buf[sl, nl, bl, pl.ds(cc * 16, 16)] = zeros16
                return c
            lax.fori_loop(0, NL * BL, zbody, 0)

        lanes = lax.iota(jnp.int32, 16)
        lane_hi = lanes >> 3
        lane_bl = lanes & 7
        gvecs = [jnp.full((16,), float(v), jnp.float32) for v in _GVALS]
        zvecs = [zeros16] * KSIZE

        def scatter_chunk(ci, slot, vals):
            n0 = ci * NL
            slotv = jnp.full((16,), slot, jnp.int32)

            def gbody(g, c):
                nlv = lane_hi + 2 * g
                nv = n0 + nlv
                act = nv < N
                p = plsc.load_gather(posv, [lane_bl, nv],
                                     mask=act).astype(jnp.int32)
                o = plsc.load_gather(othv, [lane_bl, nv],
                                     mask=act).astype(jnp.int32)
                ul = p - RADIUS
                br = p + RADIUS + 1
                ulo = o - RADIUS
                bro = o + RADIUS + 1
                in_ul = (ul >= 0) & (ul <= OUT_W) & (ulo >= 0) & (ulo <= OUT_W)
                in_br = (br >= 0) & (br <= OUT_W) & (bro >= 0) & (bro <= OUT_W)
                valid = (in_ul | in_br) & act
                for j in range(KSIZE):
                    col = ul + j
                    m = valid & (col >= 0) & (col < OUT_W)
                    plsc.store_scatter(buf, [slotv, nlv, lane_bl, col],
                                       vals[j], mask=m)
                return c

            lax.fori_loop(0, GPC, gbody, 0)

        def issue(ci, slot):
            n0 = ci * NL
            for xv in range(2):
                xcond = is_x if xv == 0 else jnp.logical_not(is_x)
                for sl in range(2):
                    @pl.when(xcond & (slot == sl) & (ci < CHUNKS - 1))
                    def _d(xv=xv, sl=sl):
                        pltpu.async_copy(
                            buf.at[sl],
                            outs[xv].at[pl.ds(n0, NL), pl.ds(b0, BL)],
                            sems[sl])

                @pl.when(xcond & (ci == CHUNKS - 1))
                def _dl(xv=xv):
                    pltpu.async_copy(
                        buf.at[1, pl.ds(0, NLAST)],
                        outs[xv].at[pl.ds(n0, NLAST), pl.ds(b0, BL)],
                        sem1)

        zero_half(0)
        scatter_chunk(0, 0, gvecs)
        issue(0, 0)
        zero_half(1)

        def body(ci, c):
            slot = lax.rem(ci, 2)

            @pl.when(ci >= 2)
            def _drain_and_restore():
                for sl in range(2):
                    @pl.when(slot == sl)
                    def _w(sl=sl):
                        pltpu.make_async_copy(
                            buf.at[sl],
                            out_x.at[pl.ds(0, NL), pl.ds(0, BL)],
                            sems[sl]).wait()

                scatter_chunk(ci - 2, slot, zvecs)

            scatter_chunk(ci, slot, gvecs)
            issue(ci, slot)
            return c

        lax.fori_loop(1, CHUNKS, body, 0)

        pltpu.make_async_copy(
            buf.at[0], out_x.at[pl.ds(0, NL), pl.ds(0, BL)], sem0).wait()
        pltpu.make_async_copy(
            buf.at[1, pl.ds(0, NLAST)],
            out_x.at[pl.ds(0, NLAST), pl.ds(0, BL)], sem1).wait()

    return k(xpl, ypl)


def kernel(lmks):
    lm_scaled = lmks * UPSCALE / STRIDE
    ox, oy = _sc_scatter(lm_scaled[..., 0], lm_scaled[..., 1])
    return ox.transpose(1, 0, 2), oy.transpose(1, 0, 2)
